# baseline (speedup 1.0000x reference)
import functools

import jax
import jax.numpy as jnp
from jax import lax
from jax.experimental import pallas as pl
from jax.experimental.pallas import tpu as pltpu
from jax.experimental.pallas import tpu_sc as plsc

_N = 10000
_E = 160000
_D = 256
_H = 128
_G = 32
_OUT = 128
_L = 4
_BN_EPS = 1e-3

_CHUNK = 128
_SUBCORES = 16
_EPAD = 163840
_IROWS = _EPAD // _CHUNK
_IROWS_PT = _IROWS // _SUBCORES
_NROWS = _N + _CHUNK
_RPT = 624
_TAIL0 = _SUBCORES * _RPT
_TAIL = _N - _TAIL0

_R = 400
_NB = _N // _R


def _sc_agg_body(h_lo, h_hi, src, dst, out_lo, out_hi,
                 spmem,
                 src_v0, src_v1, src_v2, src_v3,
                 dst_v0, dst_v1, dst_v2, dst_v3,
                 rows_v0, rows_v1,
                 sem_g, sem_s0, sem_s1, sem_i1, sem_i2, sem_i3):
    c = lax.axis_index("c")
    s = lax.axis_index("s")
    src_v = (src_v0, src_v1, src_v2, src_v3)
    dst_v = (dst_v0, dst_v1, dst_v2, dst_v3)
    rows_v = (rows_v0, rows_v1)
    sem_s = (sem_s0, sem_s1)
    sem_i = (None, sem_i1, sem_i2, sem_i3)

    def run(h_ref, out_ref):
        r0 = s * _RPT
        pltpu.sync_copy(h_ref.at[pl.ds(r0, _RPT)], spmem.at[pl.ds(r0, _RPT)])

        @pl.when(s == 0)
        def _():
            pltpu.sync_copy(h_ref.at[pl.ds(_TAIL0, _TAIL)],
                            spmem.at[pl.ds(_TAIL0, _TAIL)])

        plsc.subcore_barrier()

        def body(j, carry):
            base = (j * _SUBCORES + s) * _CHUNK
            pltpu.sync_copy(src.at[pl.ds(base, _CHUNK)], src_v[0])
            pltpu.sync_copy(dst.at[pl.ds(base, _CHUNK)], dst_v[0])
            pltpu.async_copy(h_ref.at[src_v[0]], rows_v[0], sem_g).wait()
            pltpu.sync_copy(rows_v[0], spmem.at[dst_v[0]], add=True)
            return carry

        lax.fori_loop(0, _IROWS_PT, body, 0)
        plsc.subcore_barrier()
        pltpu.sync_copy(spmem.at[pl.ds(r0, _RPT)], out_ref.at[pl.ds(r0, _RPT)])

        @pl.when(s == 0)
        def _():
            pltpu.sync_copy(spmem.at[pl.ds(_TAIL0, _TAIL)],
                            out_ref.at[pl.ds(_TAIL0, _TAIL)])

    @pl.when(c == 0)
    def _():
        run(h_lo, out_lo)

    @pl.when(c == 1)
    def _():
        run(h_hi, out_hi)


_sc_agg = pl.kernel(
    _sc_agg_body,
    out_type=(
        jax.ShapeDtypeStruct((_N, _H), jnp.float32),
        jax.ShapeDtypeStruct((_N, _H), jnp.float32),
    ),
    mesh=plsc.VectorSubcoreMesh(core_axis_name="c", subcore_axis_name="s"),
    scratch_types=[
        pltpu.VMEM_SHARED((_NROWS, _H), jnp.float32),
        pltpu.VMEM((_CHUNK,), jnp.int32),
        pltpu.VMEM((_CHUNK,), jnp.int32),
        pltpu.VMEM((_CHUNK,), jnp.int32),
        pltpu.VMEM((_CHUNK,), jnp.int32),
        pltpu.VMEM((_CHUNK,), jnp.int32),
        pltpu.VMEM((_CHUNK,), jnp.int32),
        pltpu.VMEM((_CHUNK,), jnp.int32),
        pltpu.VMEM((_CHUNK,), jnp.int32),
        pltpu.VMEM((_CHUNK, _H), jnp.float32),
        pltpu.VMEM((_CHUNK, _H), jnp.float32),
        pltpu.SemaphoreType.DMA,
        pltpu.SemaphoreType.DMA,
        pltpu.SemaphoreType.DMA,
        pltpu.SemaphoreType.DMA,
        pltpu.SemaphoreType.DMA,
        pltpu.SemaphoreType.DMA,
    ],
)


def _mlp_body(eps_ref, slo_ref, shi_ref, hlo_ref, hhi_ref,
              w1_ref, b1_ref, w2_ref, b2_ref,
              u_ref, ssum_ref, ssq_ref):
    i = pl.program_id(0)
    eps_l = eps_ref[0]
    a_lo = slo_ref[...] + eps_l * hlo_ref[...]
    a_hi = shi_ref[...] + eps_l * hhi_ref[...]
    t = jnp.dot(a_lo, w1_ref[0:_H, :], preferred_element_type=jnp.float32)
    t = t + jnp.dot(a_hi, w1_ref[_H:_D, :], preferred_element_type=jnp.float32)
    t = jnp.maximum(t + b1_ref[...], 0.0)
    u = jnp.dot(t, w2_ref[...], preferred_element_type=jnp.float32) + b2_ref[...]
    u_ref[...] = u

    @pl.when(i == 0)
    def _():
        ssum_ref[...] = jnp.zeros_like(ssum_ref)
        ssq_ref[...] = jnp.zeros_like(ssq_ref)

    ssum_ref[...] += jnp.sum(u, axis=0, keepdims=True)
    ssq_ref[...] += jnp.sum(u * u, axis=0, keepdims=True)


_mlp_call = pl.pallas_call(
    _mlp_body,
    grid=(_NB,),
    in_specs=[
        pl.BlockSpec(memory_space=pltpu.SMEM),
        pl.BlockSpec((_R, _H), lambda i: (i, 0)),
        pl.BlockSpec((_R, _H), lambda i: (i, 0)),
        pl.BlockSpec((_R, _H), lambda i: (i, 0)),
        pl.BlockSpec((_R, _H), lambda i: (i, 0)),
        pl.BlockSpec((_D, _D), lambda i: (0, 0)),
        pl.BlockSpec((1, _D), lambda i: (0, 0)),
        pl.BlockSpec((_D, _D), lambda i: (0, 0)),
        pl.BlockSpec((1, _D), lambda i: (0, 0)),
    ],
    out_specs=[
        pl.BlockSpec((_R, _D), lambda i: (i, 0)),
        pl.BlockSpec((1, _D), lambda i: (0, 0)),
        pl.BlockSpec((1, _D), lambda i: (0, 0)),
    ],
    out_shape=[
        jax.ShapeDtypeStruct((_N, _D), jnp.float32),
        jax.ShapeDtypeStruct((1, _D), jnp.float32),
        jax.ShapeDtypeStruct((1, _D), jnp.float32),
    ],
)


def _onehot(ids):
    return (ids[:, None] == lax.broadcasted_iota(jnp.int32, (_R, _G), 1)
            ).astype(jnp.float32)


def _bn_body(gid_ref, u_ref, ssum_ref, ssq_ref, gam_ref, bet_ref,
             hlo_ref, hhi_ref, g_ref):
    i = pl.program_id(0)
    mean = ssum_ref[...] * (1.0 / _N)
    var = ssq_ref[...] * (1.0 / _N) - mean * mean
    scale = gam_ref[...] * lax.rsqrt(var + _BN_EPS)
    h = jnp.maximum((u_ref[...] - mean) * scale + bet_ref[...], 0.0)
    hlo_ref[...] = h[:, 0:_H]
    hhi_ref[...] = h[:, _H:_D]

    @pl.when(i == 0)
    def _():
        g_ref[...] = jnp.zeros_like(g_ref)

    oh = _onehot(gid_ref[0, 0])
    g_ref[...] += lax.dot_general(oh, h, (((0,), (0,)), ((), ())),
                                  preferred_element_type=jnp.float32)


_bn_call = pl.pallas_call(
    _bn_body,
    grid=(_NB,),
    in_specs=[
        pl.BlockSpec((1, 1, _R), lambda i: (i, 0, 0)),
        pl.BlockSpec((_R, _D), lambda i: (i, 0)),
        pl.BlockSpec((1, _D), lambda i: (0, 0)),
        pl.BlockSpec((1, _D), lambda i: (0, 0)),
        pl.BlockSpec((1, _D), lambda i: (0, 0)),
        pl.BlockSpec((1, _D), lambda i: (0, 0)),
    ],
    out_specs=[
        pl.BlockSpec((_R, _H), lambda i: (i, 0)),
        pl.BlockSpec((_R, _H), lambda i: (i, 0)),
        pl.BlockSpec((_G, _D), lambda i: (0, 0)),
    ],
    out_shape=[
        jax.ShapeDtypeStruct((_N, _H), jnp.float32),
        jax.ShapeDtypeStruct((_N, _H), jnp.float32),
        jax.ShapeDtypeStruct((_G, _D), jnp.float32),
    ],
)


def _seg_body(gid_ref, x_ref, g_ref):
    i = pl.program_id(0)

    @pl.when(i == 0)
    def _():
        g_ref[...] = jnp.zeros_like(g_ref)

    oh = _onehot(gid_ref[0, 0])
    g_ref[...] += lax.dot_general(oh, x_ref[...], (((0,), (0,)), ((), ())),
                                  preferred_element_type=jnp.float32)


_seg_call = pl.pallas_call(
    _seg_body,
    grid=(_NB,),
    in_specs=[
        pl.BlockSpec((1, 1, _R), lambda i: (i, 0, 0)),
        pl.BlockSpec((_R, _D), lambda i: (i, 0)),
    ],
    out_specs=pl.BlockSpec((_G, _D), lambda i: (0, 0)),
    out_shape=jax.ShapeDtypeStruct((_G, _D), jnp.float32),
)


def _readout_body(g_ref, w_ref, b_ref, out_ref):
    l = pl.program_id(0)

    @pl.when(l == 0)
    def _():
        out_ref[...] = jnp.zeros_like(out_ref)

    out_ref[...] += (jnp.dot(g_ref[0], w_ref[0],
                             preferred_element_type=jnp.float32) + b_ref[0, 0])


_readout_call = pl.pallas_call(
    _readout_body,
    grid=(_L + 1,),
    in_specs=[
        pl.BlockSpec((1, _G, _D), lambda l: (l, 0, 0)),
        pl.BlockSpec((1, _D, _OUT), lambda l: (l, 0, 0)),
        pl.BlockSpec((1, 1, _OUT), lambda l: (l, 0, 0)),
    ],
    out_specs=pl.BlockSpec((_G, _OUT), lambda l: (0, 0)),
    out_shape=jax.ShapeDtypeStruct((_G, _OUT), jnp.float32),
)


def kernel(x, edge_index, graph_ids, eps, mlp_w, mlp_b,
           bn_gamma, bn_beta, lin_w, lin_b):
    pad = _EPAD - _E
    src = jnp.concatenate(
        [edge_index[0].astype(jnp.int32), jnp.zeros((pad,), jnp.int32)]
    )
    dst = jnp.concatenate(
        [edge_index[1].astype(jnp.int32),
         _N + (jnp.arange(pad, dtype=jnp.int32) % _CHUNK)]
    )
    gid3 = graph_ids.astype(jnp.int32).reshape(_NB, 1, _R)

    h_lo = x[:, :_H]
    h_hi = x[:, _H:]

    g_list = [_seg_call(gid3, x)]
    for layer in range(_L):
        s_lo, s_hi = _sc_agg(h_lo, h_hi, src, dst)
        u, ssum, ssq = _mlp_call(
            eps[layer].reshape(1), s_lo, s_hi, h_lo, h_hi,
            mlp_w[layer, 0], mlp_b[layer, 0].reshape(1, _D),
            mlp_w[layer, 1], mlp_b[layer, 1].reshape(1, _D))
        h_lo, h_hi, g = _bn_call(
            gid3, u, ssum, ssq,
            bn_gamma[layer].reshape(1, _D), bn_beta[layer].reshape(1, _D))
        g_list.append(g)

    g_all = jnp.stack(g_list)
    return _readout_call(g_all, lin_w, lin_b.reshape(_L + 1, 1, _OUT))


# speedup vs baseline: 1.4512x; 1.4512x over previous
import functools

import jax
import jax.numpy as jnp
from jax import lax
from jax.experimental import pallas as pl
from jax.experimental.pallas import tpu as pltpu
from jax.experimental.pallas import tpu_sc as plsc

_N = 10000
_E = 160000
_D = 256
_H = 128
_G = 32
_OUT = 128
_L = 4
_BN_EPS = 1e-3

_CHUNK = 128
_SUBCORES = 16
_NCHUNKS = _E // _CHUNK
_ROUNDS = (_NCHUNKS + _SUBCORES - 1) // _SUBCORES
_NROWS = _N
_RPT = 624
_TAIL0 = _SUBCORES * _RPT
_TAIL = _N - _TAIL0

_R = 400
_NB = _N // _R


def _sc_agg_body(h_lo, h_hi, src, dst, out_lo, out_hi,
                 spmem, src_v, dst_v, rows_v, sem_g):
    c = lax.axis_index("c")
    s = lax.axis_index("s")

    def run(h_ref, out_ref):
        r0 = s * _RPT
        pltpu.sync_copy(h_ref.at[pl.ds(r0, _RPT)], spmem.at[pl.ds(r0, _RPT)])

        @pl.when(s == 0)
        def _():
            pltpu.sync_copy(h_ref.at[pl.ds(_TAIL0, _TAIL)],
                            spmem.at[pl.ds(_TAIL0, _TAIL)])

        plsc.subcore_barrier()

        def body(j, carry):
            chunk = j * _SUBCORES + s

            @pl.when(chunk < _NCHUNKS)
            def _():
                base = chunk * _CHUNK
                pltpu.sync_copy(src.at[pl.ds(base, _CHUNK)], src_v)
                pltpu.sync_copy(dst.at[pl.ds(base, _CHUNK)], dst_v)
                pltpu.async_copy(h_ref.at[src_v], rows_v, sem_g).wait()
                pltpu.sync_copy(rows_v, spmem.at[dst_v], add=True)

            return carry

        lax.fori_loop(0, _ROUNDS, body, 0)
        plsc.subcore_barrier()
        pltpu.sync_copy(spmem.at[pl.ds(r0, _RPT)], out_ref.at[pl.ds(r0, _RPT)])

        @pl.when(s == 0)
        def _():
            pltpu.sync_copy(spmem.at[pl.ds(_TAIL0, _TAIL)],
                            out_ref.at[pl.ds(_TAIL0, _TAIL)])

    @pl.when(c == 0)
    def _():
        run(h_lo, out_lo)

    @pl.when(c == 1)
    def _():
        run(h_hi, out_hi)


_sc_agg = pl.kernel(
    _sc_agg_body,
    out_type=(
        jax.ShapeDtypeStruct((_N, _H), jnp.float32),
        jax.ShapeDtypeStruct((_N, _H), jnp.float32),
    ),
    mesh=plsc.VectorSubcoreMesh(core_axis_name="c", subcore_axis_name="s"),
    scratch_types=[
        pltpu.VMEM_SHARED((_NROWS, _H), jnp.float32),
        pltpu.VMEM((_CHUNK,), jnp.int32),
        pltpu.VMEM((_CHUNK,), jnp.int32),
        pltpu.VMEM((_CHUNK, _H), jnp.float32),
        pltpu.SemaphoreType.DMA,
    ],
)


def _mlp_body(eps_ref, slo_ref, shi_ref, hlo_ref, hhi_ref,
              w1_ref, b1_ref, w2_ref, b2_ref,
              u_ref, ssum_ref, ssq_ref):
    i = pl.program_id(0)
    eps_l = eps_ref[0]
    a_lo = slo_ref[...] + eps_l * hlo_ref[...]
    a_hi = shi_ref[...] + eps_l * hhi_ref[...]
    t = jnp.dot(a_lo, w1_ref[0:_H, :], preferred_element_type=jnp.float32)
    t = t + jnp.dot(a_hi, w1_ref[_H:_D, :], preferred_element_type=jnp.float32)
    t = jnp.maximum(t + b1_ref[...], 0.0)
    u = jnp.dot(t, w2_ref[...], preferred_element_type=jnp.float32) + b2_ref[...]
    u_ref[...] = u

    @pl.when(i == 0)
    def _():
        ssum_ref[...] = jnp.zeros_like(ssum_ref)
        ssq_ref[...] = jnp.zeros_like(ssq_ref)

    ssum_ref[...] += jnp.sum(u, axis=0, keepdims=True)
    ssq_ref[...] += jnp.sum(u * u, axis=0, keepdims=True)


_mlp_call = pl.pallas_call(
    _mlp_body,
    grid=(_NB,),
    in_specs=[
        pl.BlockSpec(memory_space=pltpu.SMEM),
        pl.BlockSpec((_R, _H), lambda i: (i, 0)),
        pl.BlockSpec((_R, _H), lambda i: (i, 0)),
        pl.BlockSpec((_R, _H), lambda i: (i, 0)),
        pl.BlockSpec((_R, _H), lambda i: (i, 0)),
        pl.BlockSpec((_D, _D), lambda i: (0, 0)),
        pl.BlockSpec((1, _D), lambda i: (0, 0)),
        pl.BlockSpec((_D, _D), lambda i: (0, 0)),
        pl.BlockSpec((1, _D), lambda i: (0, 0)),
    ],
    out_specs=[
        pl.BlockSpec((_R, _D), lambda i: (i, 0)),
        pl.BlockSpec((1, _D), lambda i: (0, 0)),
        pl.BlockSpec((1, _D), lambda i: (0, 0)),
    ],
    out_shape=[
        jax.ShapeDtypeStruct((_N, _D), jnp.float32),
        jax.ShapeDtypeStruct((1, _D), jnp.float32),
        jax.ShapeDtypeStruct((1, _D), jnp.float32),
    ],
)


def _onehot(ids):
    return (ids[:, None] == lax.broadcasted_iota(jnp.int32, (_R, _G), 1)
            ).astype(jnp.float32)


def _bn_body(gid_ref, u_ref, ssum_ref, ssq_ref, gam_ref, bet_ref,
             hlo_ref, hhi_ref, g_ref):
    i = pl.program_id(0)
    mean = ssum_ref[...] * (1.0 / _N)
    var = ssq_ref[...] * (1.0 / _N) - mean * mean
    scale = gam_ref[...] * lax.rsqrt(var + _BN_EPS)
    h = jnp.maximum((u_ref[...] - mean) * scale + bet_ref[...], 0.0)
    hlo_ref[...] = h[:, 0:_H]
    hhi_ref[...] = h[:, _H:_D]

    @pl.when(i == 0)
    def _():
        g_ref[...] = jnp.zeros_like(g_ref)

    oh = _onehot(gid_ref[0, 0])
    g_ref[...] += lax.dot_general(oh, h, (((0,), (0,)), ((), ())),
                                  preferred_element_type=jnp.float32)


_bn_call = pl.pallas_call(
    _bn_body,
    grid=(_NB,),
    in_specs=[
        pl.BlockSpec((1, 1, _R), lambda i: (i, 0, 0)),
        pl.BlockSpec((_R, _D), lambda i: (i, 0)),
        pl.BlockSpec((1, _D), lambda i: (0, 0)),
        pl.BlockSpec((1, _D), lambda i: (0, 0)),
        pl.BlockSpec((1, _D), lambda i: (0, 0)),
        pl.BlockSpec((1, _D), lambda i: (0, 0)),
    ],
    out_specs=[
        pl.BlockSpec((_R, _H), lambda i: (i, 0)),
        pl.BlockSpec((_R, _H), lambda i: (i, 0)),
        pl.BlockSpec((_G, _D), lambda i: (0, 0)),
    ],
    out_shape=[
        jax.ShapeDtypeStruct((_N, _H), jnp.float32),
        jax.ShapeDtypeStruct((_N, _H), jnp.float32),
        jax.ShapeDtypeStruct((_G, _D), jnp.float32),
    ],
)


def _seg_body(gid_ref, x_ref, g_ref):
    i = pl.program_id(0)

    @pl.when(i == 0)
    def _():
        g_ref[...] = jnp.zeros_like(g_ref)

    oh = _onehot(gid_ref[0, 0])
    g_ref[...] += lax.dot_general(oh, x_ref[...], (((0,), (0,)), ((), ())),
                                  preferred_element_type=jnp.float32)


_seg_call = pl.pallas_call(
    _seg_body,
    grid=(_NB,),
    in_specs=[
        pl.BlockSpec((1, 1, _R), lambda i: (i, 0, 0)),
        pl.BlockSpec((_R, _D), lambda i: (i, 0)),
    ],
    out_specs=pl.BlockSpec((_G, _D), lambda i: (0, 0)),
    out_shape=jax.ShapeDtypeStruct((_G, _D), jnp.float32),
)


def _readout_body(g_ref, w_ref, b_ref, out_ref):
    l = pl.program_id(0)

    @pl.when(l == 0)
    def _():
        out_ref[...] = jnp.zeros_like(out_ref)

    out_ref[...] += (jnp.dot(g_ref[0], w_ref[0],
                             preferred_element_type=jnp.float32) + b_ref[0, 0])


_readout_call = pl.pallas_call(
    _readout_body,
    grid=(_L + 1,),
    in_specs=[
        pl.BlockSpec((1, _G, _D), lambda l: (l, 0, 0)),
        pl.BlockSpec((1, _D, _OUT), lambda l: (l, 0, 0)),
        pl.BlockSpec((1, 1, _OUT), lambda l: (l, 0, 0)),
    ],
    out_specs=pl.BlockSpec((_G, _OUT), lambda l: (0, 0)),
    out_shape=jax.ShapeDtypeStruct((_G, _OUT), jnp.float32),
)


def kernel(x, edge_index, graph_ids, eps, mlp_w, mlp_b,
           bn_gamma, bn_beta, lin_w, lin_b):
    src = edge_index[0].astype(jnp.int32)
    dst = edge_index[1].astype(jnp.int32)
    gid3 = graph_ids.astype(jnp.int32).reshape(_NB, 1, _R)

    h_lo = x[:, :_H]
    h_hi = x[:, _H:]

    g_list = [_seg_call(gid3, x)]
    for layer in range(_L):
        s_lo, s_hi = _sc_agg(h_lo, h_hi, src, dst)
        u, ssum, ssq = _mlp_call(
            eps[layer].reshape(1), s_lo, s_hi, h_lo, h_hi,
            mlp_w[layer, 0], mlp_b[layer, 0].reshape(1, _D),
            mlp_w[layer, 1], mlp_b[layer, 1].reshape(1, _D))
        h_lo, h_hi, g = _bn_call(
            gid3, u, ssum, ssq,
            bn_gamma[layer].reshape(1, _D), bn_beta[layer].reshape(1, _D))
        g_list.append(g)

    g_all = jnp.stack(g_list)
    return _readout_call(g_all, lin_w, lin_b.reshape(_L + 1, 1, _OUT))


# speedup vs baseline: 1.5756x; 1.0857x over previous
import functools

import jax
import jax.numpy as jnp
from jax import lax
from jax.experimental import pallas as pl
from jax.experimental.pallas import tpu as pltpu
from jax.experimental.pallas import tpu_sc as plsc

_N = 10000
_E = 160000
_D = 256
_H = 128
_G = 32
_OUT = 128
_L = 4
_BN_EPS = 1e-3

_CHUNK = 128
_SUBCORES = 16
_NCHUNKS = _E // _CHUNK
_ROUNDS = (_NCHUNKS + _SUBCORES - 1) // _SUBCORES
_NROWS = _N
_RPT = 624
_TAIL0 = _SUBCORES * _RPT
_TAIL = _N - _TAIL0

_R = 400
_NB = _N // _R


def _sc_agg_body(h_lo, h_hi, src, dst, out_lo, out_hi,
                 spmem, src_v, dst_v, dst_w, rows_v, rows_w,
                 sem_g, sem_s0, sem_s1):
    c = lax.axis_index("c")
    s = lax.axis_index("s")

    def run(h_ref, out_ref):
        r0 = s * _RPT
        pltpu.sync_copy(h_ref.at[pl.ds(r0, _RPT)], spmem.at[pl.ds(r0, _RPT)])

        @pl.when(s == 0)
        def _():
            pltpu.sync_copy(h_ref.at[pl.ds(_TAIL0, _TAIL)],
                            spmem.at[pl.ds(_TAIL0, _TAIL)])

        plsc.subcore_barrier()

        def body(t, carry):
            base0 = (t * 2 * _SUBCORES + s) * _CHUNK
            pltpu.sync_copy(src.at[pl.ds(base0, _CHUNK)], src_v)
            pltpu.sync_copy(dst.at[pl.ds(base0, _CHUNK)], dst_v)
            pltpu.async_copy(h_ref.at[src_v], rows_v, sem_g).wait()
            hs0 = pltpu.async_copy(rows_v, spmem.at[dst_v], sem_s0,
                                   add=True)
            base1 = ((t * 2 + 1) * _SUBCORES + s) * _CHUNK
            pltpu.sync_copy(src.at[pl.ds(base1, _CHUNK)], src_v)
            pltpu.sync_copy(dst.at[pl.ds(base1, _CHUNK)], dst_w)
            pltpu.async_copy(h_ref.at[src_v], rows_w, sem_g).wait()
            hs0.wait()
            hs1 = pltpu.async_copy(rows_w, spmem.at[dst_w], sem_s1,
                                   add=True)
            hs1.wait()
            return carry

        lax.fori_loop(0, 39, body, 0)

        @pl.when(s < 2)
        def _():
            base = (78 * _SUBCORES + s) * _CHUNK
            pltpu.sync_copy(src.at[pl.ds(base, _CHUNK)], src_v)
            pltpu.sync_copy(dst.at[pl.ds(base, _CHUNK)], dst_v)
            pltpu.async_copy(h_ref.at[src_v], rows_v, sem_g).wait()
            pltpu.sync_copy(rows_v, spmem.at[dst_v], add=True)
        plsc.subcore_barrier()
        pltpu.sync_copy(spmem.at[pl.ds(r0, _RPT)], out_ref.at[pl.ds(r0, _RPT)])

        @pl.when(s == 0)
        def _():
            pltpu.sync_copy(spmem.at[pl.ds(_TAIL0, _TAIL)],
                            out_ref.at[pl.ds(_TAIL0, _TAIL)])

    @pl.when(c == 0)
    def _():
        run(h_lo, out_lo)

    @pl.when(c == 1)
    def _():
        run(h_hi, out_hi)


_sc_agg = pl.kernel(
    _sc_agg_body,
    out_type=(
        jax.ShapeDtypeStruct((_N, _H), jnp.float32),
        jax.ShapeDtypeStruct((_N, _H), jnp.float32),
    ),
    mesh=plsc.VectorSubcoreMesh(core_axis_name="c", subcore_axis_name="s"),
    scratch_types=[
        pltpu.VMEM_SHARED((_NROWS, _H), jnp.float32),
        pltpu.VMEM((_CHUNK,), jnp.int32),
        pltpu.VMEM((_CHUNK,), jnp.int32),
        pltpu.VMEM((_CHUNK,), jnp.int32),
        pltpu.VMEM((_CHUNK, _H), jnp.float32),
        pltpu.VMEM((_CHUNK, _H), jnp.float32),
        pltpu.SemaphoreType.DMA,
        pltpu.SemaphoreType.DMA,
        pltpu.SemaphoreType.DMA,
    ],
)


def _mlp_body(eps_ref, slo_ref, shi_ref, hlo_ref, hhi_ref,
              w1_ref, b1_ref, w2_ref, b2_ref,
              u_ref, ssum_ref, ssq_ref):
    i = pl.program_id(0)
    eps_l = eps_ref[0]
    a_lo = slo_ref[...] + eps_l * hlo_ref[...]
    a_hi = shi_ref[...] + eps_l * hhi_ref[...]
    t = jnp.dot(a_lo, w1_ref[0:_H, :], preferred_element_type=jnp.float32)
    t = t + jnp.dot(a_hi, w1_ref[_H:_D, :], preferred_element_type=jnp.float32)
    t = jnp.maximum(t + b1_ref[...], 0.0)
    u = jnp.dot(t, w2_ref[...], preferred_element_type=jnp.float32) + b2_ref[...]
    u_ref[...] = u

    @pl.when(i == 0)
    def _():
        ssum_ref[...] = jnp.zeros_like(ssum_ref)
        ssq_ref[...] = jnp.zeros_like(ssq_ref)

    ssum_ref[...] += jnp.sum(u, axis=0, keepdims=True)
    ssq_ref[...] += jnp.sum(u * u, axis=0, keepdims=True)


_mlp_call = pl.pallas_call(
    _mlp_body,
    grid=(_NB,),
    in_specs=[
        pl.BlockSpec(memory_space=pltpu.SMEM),
        pl.BlockSpec((_R, _H), lambda i: (i, 0)),
        pl.BlockSpec((_R, _H), lambda i: (i, 0)),
        pl.BlockSpec((_R, _H), lambda i: (i, 0)),
        pl.BlockSpec((_R, _H), lambda i: (i, 0)),
        pl.BlockSpec((_D, _D), lambda i: (0, 0)),
        pl.BlockSpec((1, _D), lambda i: (0, 0)),
        pl.BlockSpec((_D, _D), lambda i: (0, 0)),
        pl.BlockSpec((1, _D), lambda i: (0, 0)),
    ],
    out_specs=[
        pl.BlockSpec((_R, _D), lambda i: (i, 0)),
        pl.BlockSpec((1, _D), lambda i: (0, 0)),
        pl.BlockSpec((1, _D), lambda i: (0, 0)),
    ],
    out_shape=[
        jax.ShapeDtypeStruct((_N, _D), jnp.float32),
        jax.ShapeDtypeStruct((1, _D), jnp.float32),
        jax.ShapeDtypeStruct((1, _D), jnp.float32),
    ],
)


def _onehot(ids):
    return (ids[:, None] == lax.broadcasted_iota(jnp.int32, (_R, _G), 1)
            ).astype(jnp.float32)


def _bn_body(gid_ref, u_ref, ssum_ref, ssq_ref, gam_ref, bet_ref,
             hlo_ref, hhi_ref, g_ref):
    i = pl.program_id(0)
    mean = ssum_ref[...] * (1.0 / _N)
    var = ssq_ref[...] * (1.0 / _N) - mean * mean
    scale = gam_ref[...] * lax.rsqrt(var + _BN_EPS)
    h = jnp.maximum((u_ref[...] - mean) * scale + bet_ref[...], 0.0)
    hlo_ref[...] = h[:, 0:_H]
    hhi_ref[...] = h[:, _H:_D]

    @pl.when(i == 0)
    def _():
        g_ref[...] = jnp.zeros_like(g_ref)

    oh = _onehot(gid_ref[0, 0])
    g_ref[...] += lax.dot_general(oh, h, (((0,), (0,)), ((), ())),
                                  preferred_element_type=jnp.float32)


_bn_call = pl.pallas_call(
    _bn_body,
    grid=(_NB,),
    in_specs=[
        pl.BlockSpec((1, 1, _R), lambda i: (i, 0, 0)),
        pl.BlockSpec((_R, _D), lambda i: (i, 0)),
        pl.BlockSpec((1, _D), lambda i: (0, 0)),
        pl.BlockSpec((1, _D), lambda i: (0, 0)),
        pl.BlockSpec((1, _D), lambda i: (0, 0)),
        pl.BlockSpec((1, _D), lambda i: (0, 0)),
    ],
    out_specs=[
        pl.BlockSpec((_R, _H), lambda i: (i, 0)),
        pl.BlockSpec((_R, _H), lambda i: (i, 0)),
        pl.BlockSpec((_G, _D), lambda i: (0, 0)),
    ],
    out_shape=[
        jax.ShapeDtypeStruct((_N, _H), jnp.float32),
        jax.ShapeDtypeStruct((_N, _H), jnp.float32),
        jax.ShapeDtypeStruct((_G, _D), jnp.float32),
    ],
)


def _seg_body(gid_ref, x_ref, g_ref):
    i = pl.program_id(0)

    @pl.when(i == 0)
    def _():
        g_ref[...] = jnp.zeros_like(g_ref)

    oh = _onehot(gid_ref[0, 0])
    g_ref[...] += lax.dot_general(oh, x_ref[...], (((0,), (0,)), ((), ())),
                                  preferred_element_type=jnp.float32)


_seg_call = pl.pallas_call(
    _seg_body,
    grid=(_NB,),
    in_specs=[
        pl.BlockSpec((1, 1, _R), lambda i: (i, 0, 0)),
        pl.BlockSpec((_R, _D), lambda i: (i, 0)),
    ],
    out_specs=pl.BlockSpec((_G, _D), lambda i: (0, 0)),
    out_shape=jax.ShapeDtypeStruct((_G, _D), jnp.float32),
)


def _readout_body(g_ref, w_ref, b_ref, out_ref):
    l = pl.program_id(0)

    @pl.when(l == 0)
    def _():
        out_ref[...] = jnp.zeros_like(out_ref)

    out_ref[...] += (jnp.dot(g_ref[0], w_ref[0],
                             preferred_element_type=jnp.float32) + b_ref[0, 0])


_readout_call = pl.pallas_call(
    _readout_body,
    grid=(_L + 1,),
    in_specs=[
        pl.BlockSpec((1, _G, _D), lambda l: (l, 0, 0)),
        pl.BlockSpec((1, _D, _OUT), lambda l: (l, 0, 0)),
        pl.BlockSpec((1, 1, _OUT), lambda l: (l, 0, 0)),
    ],
    out_specs=pl.BlockSpec((_G, _OUT), lambda l: (0, 0)),
    out_shape=jax.ShapeDtypeStruct((_G, _OUT), jnp.float32),
)


def kernel(x, edge_index, graph_ids, eps, mlp_w, mlp_b,
           bn_gamma, bn_beta, lin_w, lin_b):
    src = edge_index[0].astype(jnp.int32)
    dst = edge_index[1].astype(jnp.int32)
    gid3 = graph_ids.astype(jnp.int32).reshape(_NB, 1, _R)

    h_lo = x[:, :_H]
    h_hi = x[:, _H:]

    g_list = [_seg_call(gid3, x)]
    for layer in range(_L):
        s_lo, s_hi = _sc_agg(h_lo, h_hi, src, dst)
        u, ssum, ssq = _mlp_call(
            eps[layer].reshape(1), s_lo, s_hi, h_lo, h_hi,
            mlp_w[layer, 0], mlp_b[layer, 0].reshape(1, _D),
            mlp_w[layer, 1], mlp_b[layer, 1].reshape(1, _D))
        h_lo, h_hi, g = _bn_call(
            gid3, u, ssum, ssq,
            bn_gamma[layer].reshape(1, _D), bn_beta[layer].reshape(1, _D))
        g_list.append(g)

    g_all = jnp.stack(g_list)
    return _readout_call(g_all, lin_w, lin_b.reshape(_L + 1, 1, _OUT))


# speedup vs baseline: 2.0150x; 1.2789x over previous
import functools

import jax
import jax.numpy as jnp
from jax import lax
from jax.experimental import pallas as pl
from jax.experimental.pallas import tpu as pltpu
from jax.experimental.pallas import tpu_sc as plsc

_N = 10000
_E = 160000
_D = 256
_H = 128
_G = 32
_OUT = 128
_L = 4
_BN_EPS = 1e-3

_CHUNK = 128
_SUBCORES = 16
_NCHUNKS = _E // _CHUNK
_ROUNDS = (_NCHUNKS + _SUBCORES - 1) // _SUBCORES
_NROWS = _N
_RPT = 624
_TAIL0 = _SUBCORES * _RPT
_TAIL = _N - _TAIL0

_R = 400
_NB = _N // _R


def _sc_agg_body(h_lo, h_hi, src, dst, out_lo, out_hi,
                 spmem,
                 src_v0, src_v1, src_v2, src_v3,
                 dst_v0, dst_v1, dst_v2, dst_v3,
                 rows_v, rows_w,
                 sem_g, sem_s0, sem_s1, sem_i1, sem_i2, sem_i3):
    c = lax.axis_index("c")
    s = lax.axis_index("s")
    src_vs = (src_v0, src_v1, src_v2, src_v3)
    dst_vs = (dst_v0, dst_v1, dst_v2, dst_v3)
    rows_vs = (rows_v, rows_w)
    sem_ss = (sem_s0, sem_s1)
    sem_is = (None, sem_i1, sem_i2, sem_i3)

    def run(h_ref, out_ref):
        r0 = s * _RPT
        pltpu.sync_copy(h_ref.at[pl.ds(r0, _RPT)], spmem.at[pl.ds(r0, _RPT)])

        @pl.when(s == 0)
        def _():
            pltpu.sync_copy(h_ref.at[pl.ds(_TAIL0, _TAIL)],
                            spmem.at[pl.ds(_TAIL0, _TAIL)])

        plsc.subcore_barrier()

        def body(t, carry):
            def base(u):
                return ((t * 4 + u) * _SUBCORES + s) * _CHUNK

            pltpu.sync_copy(src.at[pl.ds(base(0), _CHUNK)], src_vs[0])
            pltpu.sync_copy(dst.at[pl.ds(base(0), _CHUNK)], dst_vs[0])
            hi = [None] * 4
            for u in (1, 2, 3):
                hi[u] = (
                    pltpu.async_copy(src.at[pl.ds(base(u), _CHUNK)],
                                     src_vs[u], sem_is[u]),
                    pltpu.async_copy(dst.at[pl.ds(base(u), _CHUNK)],
                                     dst_vs[u], sem_is[u]),
                )
            hs = [None] * 4
            for u in range(4):
                if u >= 1:
                    hi[u][0].wait()
                    hi[u][1].wait()
                rv = rows_vs[u % 2]
                pltpu.async_copy(h_ref.at[src_vs[u]], rv, sem_g).wait()
                if u >= 1:
                    hs[u - 1].wait()
                hs[u] = pltpu.async_copy(rv, spmem.at[dst_vs[u]],
                                         sem_ss[u % 2], add=True)
            hs[3].wait()
            return carry

        lax.fori_loop(0, 19, body, 0)

        for j in (76, 77):
            base = (j * _SUBCORES + s) * _CHUNK
            pltpu.sync_copy(src.at[pl.ds(base, _CHUNK)], src_vs[0])
            pltpu.sync_copy(dst.at[pl.ds(base, _CHUNK)], dst_vs[0])
            pltpu.async_copy(h_ref.at[src_vs[0]], rows_v, sem_g).wait()
            pltpu.sync_copy(rows_v, spmem.at[dst_vs[0]], add=True)

        @pl.when(s < 2)
        def _():
            base = (78 * _SUBCORES + s) * _CHUNK
            pltpu.sync_copy(src.at[pl.ds(base, _CHUNK)], src_vs[0])
            pltpu.sync_copy(dst.at[pl.ds(base, _CHUNK)], dst_vs[0])
            pltpu.async_copy(h_ref.at[src_vs[0]], rows_v, sem_g).wait()
            pltpu.sync_copy(rows_v, spmem.at[dst_vs[0]], add=True)
        plsc.subcore_barrier()
        pltpu.sync_copy(spmem.at[pl.ds(r0, _RPT)], out_ref.at[pl.ds(r0, _RPT)])

        @pl.when(s == 0)
        def _():
            pltpu.sync_copy(spmem.at[pl.ds(_TAIL0, _TAIL)],
                            out_ref.at[pl.ds(_TAIL0, _TAIL)])

    @pl.when(c == 0)
    def _():
        run(h_lo, out_lo)

    @pl.when(c == 1)
    def _():
        run(h_hi, out_hi)


_sc_agg = pl.kernel(
    _sc_agg_body,
    out_type=(
        jax.ShapeDtypeStruct((_N, _H), jnp.float32),
        jax.ShapeDtypeStruct((_N, _H), jnp.float32),
    ),
    mesh=plsc.VectorSubcoreMesh(core_axis_name="c", subcore_axis_name="s"),
    scratch_types=[
        pltpu.VMEM_SHARED((_NROWS, _H), jnp.float32),
        pltpu.VMEM((_CHUNK,), jnp.int32),
        pltpu.VMEM((_CHUNK,), jnp.int32),
        pltpu.VMEM((_CHUNK,), jnp.int32),
        pltpu.VMEM((_CHUNK,), jnp.int32),
        pltpu.VMEM((_CHUNK,), jnp.int32),
        pltpu.VMEM((_CHUNK,), jnp.int32),
        pltpu.VMEM((_CHUNK,), jnp.int32),
        pltpu.VMEM((_CHUNK,), jnp.int32),
        pltpu.VMEM((_CHUNK, _H), jnp.float32),
        pltpu.VMEM((_CHUNK, _H), jnp.float32),
        pltpu.SemaphoreType.DMA,
        pltpu.SemaphoreType.DMA,
        pltpu.SemaphoreType.DMA,
        pltpu.SemaphoreType.DMA,
        pltpu.SemaphoreType.DMA,
        pltpu.SemaphoreType.DMA,
    ],
)


def _mlp_body(eps_ref, slo_ref, shi_ref, hlo_ref, hhi_ref,
              w1_ref, b1_ref, w2_ref, b2_ref,
              u_ref, ssum_ref, ssq_ref):
    i = pl.program_id(0)
    eps_l = eps_ref[0]
    a_lo = slo_ref[...] + eps_l * hlo_ref[...]
    a_hi = shi_ref[...] + eps_l * hhi_ref[...]
    t = jnp.dot(a_lo, w1_ref[0:_H, :], preferred_element_type=jnp.float32)
    t = t + jnp.dot(a_hi, w1_ref[_H:_D, :], preferred_element_type=jnp.float32)
    t = jnp.maximum(t + b1_ref[...], 0.0)
    u = jnp.dot(t, w2_ref[...], preferred_element_type=jnp.float32) + b2_ref[...]
    u_ref[...] = u

    @pl.when(i == 0)
    def _():
        ssum_ref[...] = jnp.zeros_like(ssum_ref)
        ssq_ref[...] = jnp.zeros_like(ssq_ref)

    ssum_ref[...] += jnp.sum(u, axis=0, keepdims=True)
    ssq_ref[...] += jnp.sum(u * u, axis=0, keepdims=True)


_mlp_call = pl.pallas_call(
    _mlp_body,
    grid=(_NB,),
    in_specs=[
        pl.BlockSpec(memory_space=pltpu.SMEM),
        pl.BlockSpec((_R, _H), lambda i: (i, 0)),
        pl.BlockSpec((_R, _H), lambda i: (i, 0)),
        pl.BlockSpec((_R, _H), lambda i: (i, 0)),
        pl.BlockSpec((_R, _H), lambda i: (i, 0)),
        pl.BlockSpec((_D, _D), lambda i: (0, 0)),
        pl.BlockSpec((1, _D), lambda i: (0, 0)),
        pl.BlockSpec((_D, _D), lambda i: (0, 0)),
        pl.BlockSpec((1, _D), lambda i: (0, 0)),
    ],
    out_specs=[
        pl.BlockSpec((_R, _D), lambda i: (i, 0)),
        pl.BlockSpec((1, _D), lambda i: (0, 0)),
        pl.BlockSpec((1, _D), lambda i: (0, 0)),
    ],
    out_shape=[
        jax.ShapeDtypeStruct((_N, _D), jnp.float32),
        jax.ShapeDtypeStruct((1, _D), jnp.float32),
        jax.ShapeDtypeStruct((1, _D), jnp.float32),
    ],
)


def _onehot(ids):
    return (ids[:, None] == lax.broadcasted_iota(jnp.int32, (_R, _G), 1)
            ).astype(jnp.float32)


def _bn_body(gid_ref, u_ref, ssum_ref, ssq_ref, gam_ref, bet_ref,
             hlo_ref, hhi_ref, g_ref):
    i = pl.program_id(0)
    mean = ssum_ref[...] * (1.0 / _N)
    var = ssq_ref[...] * (1.0 / _N) - mean * mean
    scale = gam_ref[...] * lax.rsqrt(var + _BN_EPS)
    h = jnp.maximum((u_ref[...] - mean) * scale + bet_ref[...], 0.0)
    hlo_ref[...] = h[:, 0:_H]
    hhi_ref[...] = h[:, _H:_D]

    @pl.when(i == 0)
    def _():
        g_ref[...] = jnp.zeros_like(g_ref)

    oh = _onehot(gid_ref[0, 0])
    g_ref[...] += lax.dot_general(oh, h, (((0,), (0,)), ((), ())),
                                  preferred_element_type=jnp.float32)


_bn_call = pl.pallas_call(
    _bn_body,
    grid=(_NB,),
    in_specs=[
        pl.BlockSpec((1, 1, _R), lambda i: (i, 0, 0)),
        pl.BlockSpec((_R, _D), lambda i: (i, 0)),
        pl.BlockSpec((1, _D), lambda i: (0, 0)),
        pl.BlockSpec((1, _D), lambda i: (0, 0)),
        pl.BlockSpec((1, _D), lambda i: (0, 0)),
        pl.BlockSpec((1, _D), lambda i: (0, 0)),
    ],
    out_specs=[
        pl.BlockSpec((_R, _H), lambda i: (i, 0)),
        pl.BlockSpec((_R, _H), lambda i: (i, 0)),
        pl.BlockSpec((_G, _D), lambda i: (0, 0)),
    ],
    out_shape=[
        jax.ShapeDtypeStruct((_N, _H), jnp.float32),
        jax.ShapeDtypeStruct((_N, _H), jnp.float32),
        jax.ShapeDtypeStruct((_G, _D), jnp.float32),
    ],
)


def _seg_body(gid_ref, x_ref, g_ref):
    i = pl.program_id(0)

    @pl.when(i == 0)
    def _():
        g_ref[...] = jnp.zeros_like(g_ref)

    oh = _onehot(gid_ref[0, 0])
    g_ref[...] += lax.dot_general(oh, x_ref[...], (((0,), (0,)), ((), ())),
                                  preferred_element_type=jnp.float32)


_seg_call = pl.pallas_call(
    _seg_body,
    grid=(_NB,),
    in_specs=[
        pl.BlockSpec((1, 1, _R), lambda i: (i, 0, 0)),
        pl.BlockSpec((_R, _D), lambda i: (i, 0)),
    ],
    out_specs=pl.BlockSpec((_G, _D), lambda i: (0, 0)),
    out_shape=jax.ShapeDtypeStruct((_G, _D), jnp.float32),
)


def _readout_body(g_ref, w_ref, b_ref, out_ref):
    l = pl.program_id(0)

    @pl.when(l == 0)
    def _():
        out_ref[...] = jnp.zeros_like(out_ref)

    out_ref[...] += (jnp.dot(g_ref[0], w_ref[0],
                             preferred_element_type=jnp.float32) + b_ref[0, 0])


_readout_call = pl.pallas_call(
    _readout_body,
    grid=(_L + 1,),
    in_specs=[
        pl.BlockSpec((1, _G, _D), lambda l: (l, 0, 0)),
        pl.BlockSpec((1, _D, _OUT), lambda l: (l, 0, 0)),
        pl.BlockSpec((1, 1, _OUT), lambda l: (l, 0, 0)),
    ],
    out_specs=pl.BlockSpec((_G, _OUT), lambda l: (0, 0)),
    out_shape=jax.ShapeDtypeStruct((_G, _OUT), jnp.float32),
)


def kernel(x, edge_index, graph_ids, eps, mlp_w, mlp_b,
           bn_gamma, bn_beta, lin_w, lin_b):
    src = edge_index[0].astype(jnp.int32)
    dst = edge_index[1].astype(jnp.int32)
    gid3 = graph_ids.astype(jnp.int32).reshape(_NB, 1, _R)

    h_lo = x[:, :_H]
    h_hi = x[:, _H:]

    g_list = [_seg_call(gid3, x)]
    for layer in range(_L):
        s_lo, s_hi = _sc_agg(h_lo, h_hi, src, dst)
        u, ssum, ssq = _mlp_call(
            eps[layer].reshape(1), s_lo, s_hi, h_lo, h_hi,
            mlp_w[layer, 0], mlp_b[layer, 0].reshape(1, _D),
            mlp_w[layer, 1], mlp_b[layer, 1].reshape(1, _D))
        h_lo, h_hi, g = _bn_call(
            gid3, u, ssum, ssq,
            bn_gamma[layer].reshape(1, _D), bn_beta[layer].reshape(1, _D))
        g_list.append(g)

    g_all = jnp.stack(g_list)
    return _readout_call(g_all, lin_w, lin_b.reshape(_L + 1, 1, _OUT))


# speedup vs baseline: 2.1634x; 1.0737x over previous
import functools

import jax
import jax.numpy as jnp
from jax import lax
from jax.experimental import pallas as pl
from jax.experimental.pallas import tpu as pltpu
from jax.experimental.pallas import tpu_sc as plsc

_N = 10000
_E = 160000
_D = 256
_H = 128
_G = 32
_OUT = 128
_L = 4
_BN_EPS = 1e-3

_CHUNK = 128
_SUBCORES = 16
_NCHUNKS = _E // _CHUNK
_ROUNDS = (_NCHUNKS + _SUBCORES - 1) // _SUBCORES
_NROWS = _N
_RPT = 624
_TAIL0 = _SUBCORES * _RPT
_TAIL = _N - _TAIL0

_R = 400
_NB = _N // _R


def _sc_agg_body(h_lo, h_hi, src, dst, out_lo, out_hi,
                 spmem,
                 src_v0, src_v1, src_v2, src_v3,
                 src_v4, src_v5, src_v6, src_v7,
                 dst_v0, dst_v1, dst_v2, dst_v3,
                 dst_v4, dst_v5, dst_v6, dst_v7,
                 rows_v, rows_w,
                 sem_g, sem_s0, sem_s1,
                 sem_i1, sem_i2, sem_i3, sem_i4,
                 sem_i5, sem_i6, sem_i7):
    c = lax.axis_index("c")
    s = lax.axis_index("s")
    src_vs = (src_v0, src_v1, src_v2, src_v3,
              src_v4, src_v5, src_v6, src_v7)
    dst_vs = (dst_v0, dst_v1, dst_v2, dst_v3,
              dst_v4, dst_v5, dst_v6, dst_v7)
    rows_vs = (rows_v, rows_w)
    sem_ss = (sem_s0, sem_s1)
    sem_is = (None, sem_i1, sem_i2, sem_i3,
              sem_i4, sem_i5, sem_i6, sem_i7)

    def run(h_ref, out_ref):
        r0 = s * _RPT
        pltpu.sync_copy(h_ref.at[pl.ds(r0, _RPT)], spmem.at[pl.ds(r0, _RPT)])

        @pl.when(s == 0)
        def _():
            pltpu.sync_copy(h_ref.at[pl.ds(_TAIL0, _TAIL)],
                            spmem.at[pl.ds(_TAIL0, _TAIL)])

        plsc.subcore_barrier()

        def block(j0, n):
            def base(u):
                return ((j0 + u) * _SUBCORES + s) * _CHUNK

            pltpu.sync_copy(src.at[pl.ds(base(0), _CHUNK)], src_vs[0])
            pltpu.sync_copy(dst.at[pl.ds(base(0), _CHUNK)], dst_vs[0])
            hi = [None] * n
            for u in range(1, n):
                hi[u] = (
                    pltpu.async_copy(src.at[pl.ds(base(u), _CHUNK)],
                                     src_vs[u], sem_is[u]),
                    pltpu.async_copy(dst.at[pl.ds(base(u), _CHUNK)],
                                     dst_vs[u], sem_is[u]),
                )
            hs = [None] * n
            for u in range(n):
                if u >= 1:
                    hi[u][0].wait()
                    hi[u][1].wait()
                rv = rows_vs[u % 2]
                pltpu.async_copy(h_ref.at[src_vs[u]], rv, sem_g).wait()
                if u >= 1:
                    hs[u - 1].wait()
                hs[u] = pltpu.async_copy(rv, spmem.at[dst_vs[u]],
                                         sem_ss[u % 2], add=True)
            hs[n - 1].wait()

        def body(t, carry):
            block(t * 8, 8)
            return carry

        lax.fori_loop(0, 9, body, 0)
        block(72, 6)

        @pl.when(s < 2)
        def _():
            base = (78 * _SUBCORES + s) * _CHUNK
            pltpu.sync_copy(src.at[pl.ds(base, _CHUNK)], src_vs[0])
            pltpu.sync_copy(dst.at[pl.ds(base, _CHUNK)], dst_vs[0])
            pltpu.async_copy(h_ref.at[src_vs[0]], rows_v, sem_g).wait()
            pltpu.sync_copy(rows_v, spmem.at[dst_vs[0]], add=True)
        plsc.subcore_barrier()
        pltpu.sync_copy(spmem.at[pl.ds(r0, _RPT)], out_ref.at[pl.ds(r0, _RPT)])

        @pl.when(s == 0)
        def _():
            pltpu.sync_copy(spmem.at[pl.ds(_TAIL0, _TAIL)],
                            out_ref.at[pl.ds(_TAIL0, _TAIL)])

    @pl.when(c == 0)
    def _():
        run(h_lo, out_lo)

    @pl.when(c == 1)
    def _():
        run(h_hi, out_hi)


_sc_agg = pl.kernel(
    _sc_agg_body,
    out_type=(
        jax.ShapeDtypeStruct((_N, _H), jnp.float32),
        jax.ShapeDtypeStruct((_N, _H), jnp.float32),
    ),
    mesh=plsc.VectorSubcoreMesh(core_axis_name="c", subcore_axis_name="s"),
    scratch_types=[
        pltpu.VMEM_SHARED((_NROWS, _H), jnp.float32),
        *([pltpu.VMEM((_CHUNK,), jnp.int32)] * 16),
        pltpu.VMEM((_CHUNK, _H), jnp.float32),
        pltpu.VMEM((_CHUNK, _H), jnp.float32),
        *([pltpu.SemaphoreType.DMA] * 10),
    ],
)


def _mlp_body(eps_ref, slo_ref, shi_ref, hlo_ref, hhi_ref,
              w1_ref, b1_ref, w2_ref, b2_ref,
              u_ref, ssum_ref, ssq_ref):
    i = pl.program_id(0)
    eps_l = eps_ref[0]
    a_lo = slo_ref[...] + eps_l * hlo_ref[...]
    a_hi = shi_ref[...] + eps_l * hhi_ref[...]
    t = jnp.dot(a_lo, w1_ref[0:_H, :], preferred_element_type=jnp.float32)
    t = t + jnp.dot(a_hi, w1_ref[_H:_D, :], preferred_element_type=jnp.float32)
    t = jnp.maximum(t + b1_ref[...], 0.0)
    u = jnp.dot(t, w2_ref[...], preferred_element_type=jnp.float32) + b2_ref[...]
    u_ref[...] = u

    @pl.when(i == 0)
    def _():
        ssum_ref[...] = jnp.zeros_like(ssum_ref)
        ssq_ref[...] = jnp.zeros_like(ssq_ref)

    ssum_ref[...] += jnp.sum(u, axis=0, keepdims=True)
    ssq_ref[...] += jnp.sum(u * u, axis=0, keepdims=True)


_mlp_call = pl.pallas_call(
    _mlp_body,
    grid=(_NB,),
    in_specs=[
        pl.BlockSpec(memory_space=pltpu.SMEM),
        pl.BlockSpec((_R, _H), lambda i: (i, 0)),
        pl.BlockSpec((_R, _H), lambda i: (i, 0)),
        pl.BlockSpec((_R, _H), lambda i: (i, 0)),
        pl.BlockSpec((_R, _H), lambda i: (i, 0)),
        pl.BlockSpec((_D, _D), lambda i: (0, 0)),
        pl.BlockSpec((1, _D), lambda i: (0, 0)),
        pl.BlockSpec((_D, _D), lambda i: (0, 0)),
        pl.BlockSpec((1, _D), lambda i: (0, 0)),
    ],
    out_specs=[
        pl.BlockSpec((_R, _D), lambda i: (i, 0)),
        pl.BlockSpec((1, _D), lambda i: (0, 0)),
        pl.BlockSpec((1, _D), lambda i: (0, 0)),
    ],
    out_shape=[
        jax.ShapeDtypeStruct((_N, _D), jnp.float32),
        jax.ShapeDtypeStruct((1, _D), jnp.float32),
        jax.ShapeDtypeStruct((1, _D), jnp.float32),
    ],
)


def _onehot(ids):
    return (ids[:, None] == lax.broadcasted_iota(jnp.int32, (_R, _G), 1)
            ).astype(jnp.float32)


def _bn_body(gid_ref, u_ref, ssum_ref, ssq_ref, gam_ref, bet_ref,
             hlo_ref, hhi_ref, g_ref):
    i = pl.program_id(0)
    mean = ssum_ref[...] * (1.0 / _N)
    var = ssq_ref[...] * (1.0 / _N) - mean * mean
    scale = gam_ref[...] * lax.rsqrt(var + _BN_EPS)
    h = jnp.maximum((u_ref[...] - mean) * scale + bet_ref[...], 0.0)
    hlo_ref[...] = h[:, 0:_H]
    hhi_ref[...] = h[:, _H:_D]

    @pl.when(i == 0)
    def _():
        g_ref[...] = jnp.zeros_like(g_ref)

    oh = _onehot(gid_ref[0, 0])
    g_ref[...] += lax.dot_general(oh, h, (((0,), (0,)), ((), ())),
                                  preferred_element_type=jnp.float32)


_bn_call = pl.pallas_call(
    _bn_body,
    grid=(_NB,),
    in_specs=[
        pl.BlockSpec((1, 1, _R), lambda i: (i, 0, 0)),
        pl.BlockSpec((_R, _D), lambda i: (i, 0)),
        pl.BlockSpec((1, _D), lambda i: (0, 0)),
        pl.BlockSpec((1, _D), lambda i: (0, 0)),
        pl.BlockSpec((1, _D), lambda i: (0, 0)),
        pl.BlockSpec((1, _D), lambda i: (0, 0)),
    ],
    out_specs=[
        pl.BlockSpec((_R, _H), lambda i: (i, 0)),
        pl.BlockSpec((_R, _H), lambda i: (i, 0)),
        pl.BlockSpec((_G, _D), lambda i: (0, 0)),
    ],
    out_shape=[
        jax.ShapeDtypeStruct((_N, _H), jnp.float32),
        jax.ShapeDtypeStruct((_N, _H), jnp.float32),
        jax.ShapeDtypeStruct((_G, _D), jnp.float32),
    ],
)


def _seg_body(gid_ref, x_ref, g_ref):
    i = pl.program_id(0)

    @pl.when(i == 0)
    def _():
        g_ref[...] = jnp.zeros_like(g_ref)

    oh = _onehot(gid_ref[0, 0])
    g_ref[...] += lax.dot_general(oh, x_ref[...], (((0,), (0,)), ((), ())),
                                  preferred_element_type=jnp.float32)


_seg_call = pl.pallas_call(
    _seg_body,
    grid=(_NB,),
    in_specs=[
        pl.BlockSpec((1, 1, _R), lambda i: (i, 0, 0)),
        pl.BlockSpec((_R, _D), lambda i: (i, 0)),
    ],
    out_specs=pl.BlockSpec((_G, _D), lambda i: (0, 0)),
    out_shape=jax.ShapeDtypeStruct((_G, _D), jnp.float32),
)


def _readout_body(g_ref, w_ref, b_ref, out_ref):
    l = pl.program_id(0)

    @pl.when(l == 0)
    def _():
        out_ref[...] = jnp.zeros_like(out_ref)

    out_ref[...] += (jnp.dot(g_ref[0], w_ref[0],
                             preferred_element_type=jnp.float32) + b_ref[0, 0])


_readout_call = pl.pallas_call(
    _readout_body,
    grid=(_L + 1,),
    in_specs=[
        pl.BlockSpec((1, _G, _D), lambda l: (l, 0, 0)),
        pl.BlockSpec((1, _D, _OUT), lambda l: (l, 0, 0)),
        pl.BlockSpec((1, 1, _OUT), lambda l: (l, 0, 0)),
    ],
    out_specs=pl.BlockSpec((_G, _OUT), lambda l: (0, 0)),
    out_shape=jax.ShapeDtypeStruct((_G, _OUT), jnp.float32),
)


def kernel(x, edge_index, graph_ids, eps, mlp_w, mlp_b,
           bn_gamma, bn_beta, lin_w, lin_b):
    src = edge_index[0].astype(jnp.int32)
    dst = edge_index[1].astype(jnp.int32)
    gid3 = graph_ids.astype(jnp.int32).reshape(_NB, 1, _R)

    h_lo = x[:, :_H]
    h_hi = x[:, _H:]

    g_list = [_seg_call(gid3, x)]
    for layer in range(_L):
        s_lo, s_hi = _sc_agg(h_lo, h_hi, src, dst)
        u, ssum, ssq = _mlp_call(
            eps[layer].reshape(1), s_lo, s_hi, h_lo, h_hi,
            mlp_w[layer, 0], mlp_b[layer, 0].reshape(1, _D),
            mlp_w[layer, 1], mlp_b[layer, 1].reshape(1, _D))
        h_lo, h_hi, g = _bn_call(
            gid3, u, ssum, ssq,
            bn_gamma[layer].reshape(1, _D), bn_beta[layer].reshape(1, _D))
        g_list.append(g)

    g_all = jnp.stack(g_list)
    return _readout_call(g_all, lin_w, lin_b.reshape(_L + 1, 1, _OUT))


# speedup vs baseline: 2.2502x; 1.0401x over previous
import functools

import jax
import jax.numpy as jnp
from jax import lax
from jax.experimental import pallas as pl
from jax.experimental.pallas import tpu as pltpu
from jax.experimental.pallas import tpu_sc as plsc

_N = 10000
_E = 160000
_D = 256
_H = 128
_G = 32
_OUT = 128
_L = 4
_BN_EPS = 1e-3

_CHUNK = 128
_SUBCORES = 16
_NCHUNKS = _E // _CHUNK
_ROUNDS = (_NCHUNKS + _SUBCORES - 1) // _SUBCORES
_NROWS = _N
_RPT = 624
_TAIL0 = _SUBCORES * _RPT
_TAIL = _N - _TAIL0

_R = 400
_NB = _N // _R


def _sc_agg_body(h_lo, h_hi, src, dst, out_lo, out_hi,
                 spmem,
                 src_v0, src_v1, src_v2, src_v3,
                 src_v4, src_v5, src_v6, src_v7,
                 dst_v0, dst_v1, dst_v2, dst_v3,
                 dst_v4, dst_v5, dst_v6, dst_v7,
                 rows_v, rows_w,
                 sem_g, sem_s0, sem_s1,
                 sem_i1, sem_i2, sem_i3, sem_i4,
                 sem_i5, sem_i6, sem_i7):
    c = lax.axis_index("c")
    s = lax.axis_index("s")
    src_vs = (src_v0, src_v1, src_v2, src_v3,
              src_v4, src_v5, src_v6, src_v7)
    dst_vs = (dst_v0, dst_v1, dst_v2, dst_v3,
              dst_v4, dst_v5, dst_v6, dst_v7)
    rows_vs = (rows_v, rows_w)
    sem_ss = (sem_s0, sem_s1)
    sem_is = (None, sem_i1, sem_i2, sem_i3,
              sem_i4, sem_i5, sem_i6, sem_i7)

    def run(h_ref, out_ref):
        r0 = s * _RPT
        pltpu.sync_copy(h_ref.at[pl.ds(r0, _RPT)], spmem.at[pl.ds(r0, _RPT)])

        @pl.when(s == 0)
        def _():
            pltpu.sync_copy(h_ref.at[pl.ds(_TAIL0, _TAIL)],
                            spmem.at[pl.ds(_TAIL0, _TAIL)])

        plsc.subcore_barrier()

        def block(j0, n):
            def base(u):
                return ((j0 + u) * _SUBCORES + s) * _CHUNK

            pltpu.sync_copy(src.at[pl.ds(base(0), _CHUNK)], src_vs[0])
            pltpu.sync_copy(dst.at[pl.ds(base(0), _CHUNK)], dst_vs[0])
            hi = [None] * n
            for u in range(1, n):
                hi[u] = (
                    pltpu.async_copy(src.at[pl.ds(base(u), _CHUNK)],
                                     src_vs[u], sem_is[u]),
                    pltpu.async_copy(dst.at[pl.ds(base(u), _CHUNK)],
                                     dst_vs[u], sem_is[u]),
                )
            hs = [None] * n
            for u in range(n):
                if u >= 1:
                    hi[u][0].wait()
                    hi[u][1].wait()
                rv = rows_vs[u % 2]
                pltpu.async_copy(h_ref.at[src_vs[u]], rv, sem_g).wait()
                if u >= 1:
                    hs[u - 1].wait()
                hs[u] = pltpu.async_copy(rv, spmem.at[dst_vs[u]],
                                         sem_ss[u % 2], add=True)
            hs[n - 1].wait()

        def body(t, carry):
            block(t * 8, 8)
            return carry

        lax.fori_loop(0, 9, body, 0)
        block(72, 6)

        @pl.when(s < 2)
        def _():
            base = (78 * _SUBCORES + s) * _CHUNK
            pltpu.sync_copy(src.at[pl.ds(base, _CHUNK)], src_vs[0])
            pltpu.sync_copy(dst.at[pl.ds(base, _CHUNK)], dst_vs[0])
            pltpu.async_copy(h_ref.at[src_vs[0]], rows_v, sem_g).wait()
            pltpu.sync_copy(rows_v, spmem.at[dst_vs[0]], add=True)
        plsc.subcore_barrier()
        pltpu.sync_copy(spmem.at[pl.ds(r0, _RPT)], out_ref.at[pl.ds(r0, _RPT)])

        @pl.when(s == 0)
        def _():
            pltpu.sync_copy(spmem.at[pl.ds(_TAIL0, _TAIL)],
                            out_ref.at[pl.ds(_TAIL0, _TAIL)])

    @pl.when(c == 0)
    def _():
        run(h_lo, out_lo)

    @pl.when(c == 1)
    def _():
        run(h_hi, out_hi)


_sc_agg = pl.kernel(
    _sc_agg_body,
    out_type=(
        jax.ShapeDtypeStruct((_N, _H), jnp.float32),
        jax.ShapeDtypeStruct((_N, _H), jnp.float32),
    ),
    mesh=plsc.VectorSubcoreMesh(core_axis_name="c", subcore_axis_name="s"),
    scratch_types=[
        pltpu.VMEM_SHARED((_NROWS, _H), jnp.float32),
        *([pltpu.VMEM((_CHUNK,), jnp.int32)] * 16),
        pltpu.VMEM((_CHUNK, _H), jnp.float32),
        pltpu.VMEM((_CHUNK, _H), jnp.float32),
        *([pltpu.SemaphoreType.DMA] * 10),
    ],
)


def _layer_body(eps_ref, slo_ref, shi_ref, hlo_ref, hhi_ref,
                w1_ref, b1_ref, w2_ref, b2_ref, gam_ref, bet_ref, gid_ref,
                hlo_out, hhi_out, g_ref, u_scr, ssum_scr, ssq_scr):
    p = pl.program_id(0)
    i = pl.program_id(1)

    @pl.when(p == 0)
    def _():
        eps_l = eps_ref[0]
        a_lo = slo_ref[...] + eps_l * hlo_ref[...]
        a_hi = shi_ref[...] + eps_l * hhi_ref[...]
        t = jnp.dot(a_lo, w1_ref[0:_H, :], preferred_element_type=jnp.float32)
        t = t + jnp.dot(a_hi, w1_ref[_H:_D, :],
                        preferred_element_type=jnp.float32)
        t = jnp.maximum(t + b1_ref[...], 0.0)
        u = (jnp.dot(t, w2_ref[...], preferred_element_type=jnp.float32)
             + b2_ref[...])
        u_scr[pl.ds(i * _R, _R), :] = u

        @pl.when(i == 0)
        def _():
            ssum_scr[...] = jnp.zeros_like(ssum_scr)
            ssq_scr[...] = jnp.zeros_like(ssq_scr)

        ssum_scr[...] += jnp.sum(u, axis=0, keepdims=True)
        ssq_scr[...] += jnp.sum(u * u, axis=0, keepdims=True)

    @pl.when(p == 1)
    def _():
        mean = ssum_scr[...] * (1.0 / _N)
        var = ssq_scr[...] * (1.0 / _N) - mean * mean
        scale = gam_ref[...] * lax.rsqrt(var + _BN_EPS)
        u = u_scr[pl.ds(i * _R, _R), :]
        h = jnp.maximum((u - mean) * scale + bet_ref[...], 0.0)
        hlo_out[...] = h[:, 0:_H]
        hhi_out[...] = h[:, _H:_D]

        @pl.when(i == 0)
        def _():
            g_ref[...] = jnp.zeros_like(g_ref)

        oh = _onehot(gid_ref[0, 0])
        g_ref[...] += lax.dot_general(oh, h, (((0,), (0,)), ((), ())),
                                      preferred_element_type=jnp.float32)


_layer_call = pl.pallas_call(
    _layer_body,
    grid=(2, _NB),
    in_specs=[
        pl.BlockSpec(memory_space=pltpu.SMEM),
        pl.BlockSpec((_R, _H), lambda p, i: (i * (1 - p), 0)),
        pl.BlockSpec((_R, _H), lambda p, i: (i * (1 - p), 0)),
        pl.BlockSpec((_R, _H), lambda p, i: (i * (1 - p), 0)),
        pl.BlockSpec((_R, _H), lambda p, i: (i * (1 - p), 0)),
        pl.BlockSpec((_D, _D), lambda p, i: (0, 0)),
        pl.BlockSpec((1, _D), lambda p, i: (0, 0)),
        pl.BlockSpec((_D, _D), lambda p, i: (0, 0)),
        pl.BlockSpec((1, _D), lambda p, i: (0, 0)),
        pl.BlockSpec((1, _D), lambda p, i: (0, 0)),
        pl.BlockSpec((1, _D), lambda p, i: (0, 0)),
        pl.BlockSpec((1, 1, _R), lambda p, i: (i * p, 0, 0)),
    ],
    out_specs=[
        pl.BlockSpec((_R, _H), lambda p, i: (i * p, 0)),
        pl.BlockSpec((_R, _H), lambda p, i: (i * p, 0)),
        pl.BlockSpec((_G, _D), lambda p, i: (0, 0)),
    ],
    out_shape=[
        jax.ShapeDtypeStruct((_N, _H), jnp.float32),
        jax.ShapeDtypeStruct((_N, _H), jnp.float32),
        jax.ShapeDtypeStruct((_G, _D), jnp.float32),
    ],
    scratch_shapes=[
        pltpu.VMEM((_N, _D), jnp.float32),
        pltpu.VMEM((1, _D), jnp.float32),
        pltpu.VMEM((1, _D), jnp.float32),
    ],
)


def _onehot(ids):
    return (ids[:, None] == lax.broadcasted_iota(jnp.int32, (_R, _G), 1)
            ).astype(jnp.float32)


def _seg_body(gid_ref, x_ref, g_ref):
    i = pl.program_id(0)

    @pl.when(i == 0)
    def _():
        g_ref[...] = jnp.zeros_like(g_ref)

    oh = _onehot(gid_ref[0, 0])
    g_ref[...] += lax.dot_general(oh, x_ref[...], (((0,), (0,)), ((), ())),
                                  preferred_element_type=jnp.float32)


_seg_call = pl.pallas_call(
    _seg_body,
    grid=(_NB,),
    in_specs=[
        pl.BlockSpec((1, 1, _R), lambda i: (i, 0, 0)),
        pl.BlockSpec((_R, _D), lambda i: (i, 0)),
    ],
    out_specs=pl.BlockSpec((_G, _D), lambda i: (0, 0)),
    out_shape=jax.ShapeDtypeStruct((_G, _D), jnp.float32),
)


def _readout_body(g_ref, w_ref, b_ref, out_ref):
    l = pl.program_id(0)

    @pl.when(l == 0)
    def _():
        out_ref[...] = jnp.zeros_like(out_ref)

    out_ref[...] += (jnp.dot(g_ref[0], w_ref[0],
                             preferred_element_type=jnp.float32) + b_ref[0, 0])


_readout_call = pl.pallas_call(
    _readout_body,
    grid=(_L + 1,),
    in_specs=[
        pl.BlockSpec((1, _G, _D), lambda l: (l, 0, 0)),
        pl.BlockSpec((1, _D, _OUT), lambda l: (l, 0, 0)),
        pl.BlockSpec((1, 1, _OUT), lambda l: (l, 0, 0)),
    ],
    out_specs=pl.BlockSpec((_G, _OUT), lambda l: (0, 0)),
    out_shape=jax.ShapeDtypeStruct((_G, _OUT), jnp.float32),
)


def kernel(x, edge_index, graph_ids, eps, mlp_w, mlp_b,
           bn_gamma, bn_beta, lin_w, lin_b):
    src = edge_index[0].astype(jnp.int32)
    dst = edge_index[1].astype(jnp.int32)
    gid3 = graph_ids.astype(jnp.int32).reshape(_NB, 1, _R)

    h_lo = x[:, :_H]
    h_hi = x[:, _H:]

    g_list = [_seg_call(gid3, x)]
    for layer in range(_L):
        s_lo, s_hi = _sc_agg(h_lo, h_hi, src, dst)
        h_lo, h_hi, g = _layer_call(
            eps[layer].reshape(1), s_lo, s_hi, h_lo, h_hi,
            mlp_w[layer, 0], mlp_b[layer, 0].reshape(1, _D),
            mlp_w[layer, 1], mlp_b[layer, 1].reshape(1, _D),
            bn_gamma[layer].reshape(1, _D), bn_beta[layer].reshape(1, _D),
            gid3)
        g_list.append(g)

    g_all = jnp.stack(g_list)
    return _readout_call(g_all, lin_w, lin_b.reshape(_L + 1, 1, _OUT))
